# Initial kernel scaffold; baseline (speedup 1.0000x reference)
import jax
import jax.numpy as jnp
from jax.experimental import pallas as pl


def kernel(token_ids, E):
    raise NotImplementedError("write your pallas kernel here")


# speedup vs baseline: 2.9748x; 2.9748x over previous
import functools

import jax
import jax.numpy as jnp
from jax import lax
from jax.experimental import pallas as pl
from jax.experimental.pallas import tpu as pltpu
from jax.experimental.pallas import tpu_sc as plsc

_NC = 2
_NS = 16
_NW = _NC * _NS

_D = 128
_CHUNK = 128


def _make_gather(B: int):
    assert B % (_NW * _CHUNK) == 0
    b_per_w = B // _NW
    n_chunks = b_per_w // _CHUNK
    mesh = plsc.VectorSubcoreMesh(core_axis_name="c", subcore_axis_name="s")

    @functools.partial(
        pl.kernel,
        mesh=mesh,
        out_type=jax.ShapeDtypeStruct((B, _D), jnp.float32),
        scratch_types=[
            pltpu.VMEM((b_per_w,), jnp.int32),
            pltpu.VMEM((_CHUNK, _D), jnp.float32),
            pltpu.SemaphoreType.DMA,
        ],
    )
    def k(table_hbm, idx_hbm, out_hbm, idx_v, rows_v, sem):
        wid = lax.axis_index("s") * _NC + lax.axis_index("c")
        base = wid * b_per_w
        pltpu.sync_copy(idx_hbm.at[pl.ds(base, b_per_w)], idx_v)

        def body(j, _):
            off = j * _CHUNK
            pltpu.async_copy(
                table_hbm.at[idx_v.at[pl.ds(off, _CHUNK)]], rows_v, sem
            ).wait()
            pltpu.sync_copy(rows_v, out_hbm.at[pl.ds(base + off, _CHUNK)])
            return ()

        lax.fori_loop(0, n_chunks, body, (), unroll=False)

    return k


def kernel(token_ids, E):
    shape = token_ids.shape
    idx = token_ids.reshape(-1).astype(jnp.int32)
    out = _make_gather(idx.shape[0])(E, idx)
    return out.reshape(*shape, _D)


# speedup vs baseline: 3.3287x; 1.1189x over previous
import functools

import jax
import jax.numpy as jnp
from jax import lax
from jax.experimental import pallas as pl
from jax.experimental.pallas import tpu as pltpu
from jax.experimental.pallas import tpu_sc as plsc

_NC = 2
_NS = 16
_NW = _NC * _NS

_D = 128
_CHUNK = 128
_NBUF = 5


def _make_gather(B: int):
    assert B % (_NW * _CHUNK) == 0
    b_per_w = B // _NW
    n_chunks = b_per_w // _CHUNK
    assert n_chunks % _NBUF == 0 and n_chunks // _NBUF >= 2
    n_groups = n_chunks // _NBUF
    mesh = plsc.VectorSubcoreMesh(core_axis_name="c", subcore_axis_name="s")

    @functools.partial(
        pl.kernel,
        mesh=mesh,
        out_type=jax.ShapeDtypeStruct((B, _D), jnp.float32),
        scratch_types=[
            pltpu.VMEM((b_per_w,), jnp.int32),
            pltpu.VMEM((_NBUF, _CHUNK, _D), jnp.float32),
            pltpu.SemaphoreType.DMA((_NBUF,)),
            pltpu.SemaphoreType.DMA((_NBUF,)),
        ],
    )
    def k(table_hbm, idx_hbm, out_hbm, idx_v, rows_v, gsem, osem):
        wid = lax.axis_index("s") * _NC + lax.axis_index("c")
        base = wid * b_per_w
        pltpu.sync_copy(idx_hbm.at[pl.ds(base, b_per_w)], idx_v)

        def start_gather(j, b):
            pltpu.async_copy(
                table_hbm.at[idx_v.at[pl.ds(j * _CHUNK, _CHUNK)]],
                rows_v.at[b],
                gsem.at[b],
            )

        def wait_gather(b):
            pltpu.make_async_copy(
                table_hbm.at[idx_v.at[pl.ds(0, _CHUNK)]],
                rows_v.at[b],
                gsem.at[b],
            ).wait()

        def start_out(j, b):
            pltpu.async_copy(
                rows_v.at[b],
                out_hbm.at[pl.ds(base + j * _CHUNK, _CHUNK)],
                osem.at[b],
            )

        def wait_out(b):
            pltpu.make_async_copy(
                rows_v.at[b],
                out_hbm.at[pl.ds(base, _CHUNK)],
                osem.at[b],
            ).wait()

        start_gather(0, 0)
        for b in range(_NBUF):
            j = b
            if b == _NBUF - 1:
                wait_out(0)
            start_gather(j + 1, (b + 1) % _NBUF)
            wait_gather(b)
            start_out(j, b)

        def group(g, _):
            for b in range(_NBUF):
                j = g * _NBUF + b
                wait_out((b + 1) % _NBUF)
                start_gather(j + 1, (b + 1) % _NBUF)
                wait_gather(b)
                start_out(j, b)
            return ()

        lax.fori_loop(1, n_groups - 1, group, (), unroll=False)

        for b in range(_NBUF):
            j = (n_groups - 1) * _NBUF + b
            if b != _NBUF - 1:
                wait_out((b + 1) % _NBUF)
                start_gather(j + 1, (b + 1) % _NBUF)
            wait_gather(b)
            start_out(j, b)

        for b in range(_NBUF):
            wait_out(b)

    return k


def kernel(token_ids, E):
    shape = token_ids.shape
    idx = token_ids.reshape(-1).astype(jnp.int32)
    out = _make_gather(idx.shape[0])(E, idx)
    return out.reshape(*shape, _D)


# speedup vs baseline: 5.4995x; 1.6521x over previous
import functools

import jax
import jax.numpy as jnp
from jax import lax
from jax.experimental import pallas as pl
from jax.experimental.pallas import tpu as pltpu
from jax.experimental.pallas import tpu_sc as plsc

_NC = 2
_NS = 16
_NW = _NC * _NS

_D = 128
_NBUF = 4


def _make_gather(S: int, T: int):
    assert S % _NW == 0
    s_per_w = S // _NW
    assert s_per_w % _NBUF == 0 and s_per_w // _NBUF >= 2
    n_groups = s_per_w // _NBUF
    mesh = plsc.VectorSubcoreMesh(core_axis_name="c", subcore_axis_name="s")

    @functools.partial(
        pl.kernel,
        mesh=mesh,
        out_type=jax.ShapeDtypeStruct((S, T, _D), jnp.float32),
        scratch_types=[
            pltpu.VMEM((s_per_w, T), jnp.int32),
            pltpu.VMEM((_NBUF, T, _D), jnp.float32),
            pltpu.SemaphoreType.DMA((_NBUF,)),
            pltpu.SemaphoreType.DMA((_NBUF,)),
        ],
    )
    def k(table_hbm, idx_hbm, out_hbm, idx_v, rows_v, gsem, osem):
        wid = lax.axis_index("s") * _NC + lax.axis_index("c")
        base = wid * s_per_w
        pltpu.sync_copy(idx_hbm.at[pl.ds(base, s_per_w)], idx_v)

        def start_gather(j, b):
            pltpu.async_copy(
                table_hbm.at[idx_v.at[j]], rows_v.at[b], gsem.at[b]
            )

        def wait_gather(b):
            pltpu.make_async_copy(
                table_hbm.at[idx_v.at[0]], rows_v.at[b], gsem.at[b]
            ).wait()

        def start_out(j, b):
            pltpu.async_copy(rows_v.at[b], out_hbm.at[base + j], osem.at[b])

        def wait_out(b):
            pltpu.make_async_copy(
                rows_v.at[b], out_hbm.at[base], osem.at[b]
            ).wait()

        start_gather(0, 0)
        for b in range(_NBUF):
            if b == _NBUF - 1:
                wait_out(0)
            start_gather(b + 1, (b + 1) % _NBUF)
            wait_gather(b)
            start_out(b, b)

        def group(g, _):
            for b in range(_NBUF):
                j = g * _NBUF + b
                wait_out((b + 1) % _NBUF)
                start_gather(j + 1, (b + 1) % _NBUF)
                wait_gather(b)
                start_out(j, b)
            return ()

        lax.fori_loop(1, n_groups - 1, group, (), unroll=False)

        for b in range(_NBUF):
            j = (n_groups - 1) * _NBUF + b
            if b != _NBUF - 1:
                wait_out((b + 1) % _NBUF)
                start_gather(j + 1, (b + 1) % _NBUF)
            wait_gather(b)
            start_out(j, b)

        for b in range(_NBUF):
            wait_out(b)

    return k


def kernel(token_ids, E):
    S, T = token_ids.shape
    return _make_gather(S, T)(E, token_ids.astype(jnp.int32))
